# baseline (speedup 1.0000x reference)
import functools

import jax
import jax.numpy as jnp
from jax import lax
from jax.experimental import pallas as pl
from jax.experimental.pallas import tpu as pltpu
from jax.experimental.pallas import tpu_sc as plsc

K = 10
N_ENT = 100000
DIM = 64
B = 1024

BS_T = 256
ENT_T = 512
LANES = 128
SUBS = ENT_T // LANES
ENT_PAD = 100352
NJ = ENT_PAD // ENT_T
NEG = -3.0e38
BIG = 1 << 30


def _topk_body(hr_ref, e_ref, out_v_ref, out_i_ref, best_v, best_i, cand_v, cand_i):
    j = pl.program_id(1)

    @pl.when(j == 0)
    def _init():
        for t in range(K):
            best_v[t] = jnp.full((BS_T, LANES), NEG, jnp.float32)
            best_i[t] = jnp.zeros((BS_T, LANES), jnp.int32)

    hr = hr_ref[...]
    e = e_ref[...]
    scores = lax.dot_general(
        hr, e, (((1,), (1,)), ((), ())),
        preferred_element_type=jnp.float32,
    )

    lane_iota = lax.broadcasted_iota(jnp.int32, (BS_T, LANES), 1)
    for sub in range(SUBS):
        v = scores[:, sub * LANES:(sub + 1) * LANES]
        col0 = j * ENT_T + sub * LANES
        vi = lane_iota + col0
        v = jnp.where(vi < N_ENT, v, NEG)
        for t in range(K):
            bv = best_v[t]
            bi = best_i[t]
            gt = v > bv
            best_v[t] = jnp.where(gt, v, bv)
            best_i[t] = jnp.where(gt, vi, bi)
            v = jnp.where(gt, bv, v)
            vi = jnp.where(gt, bi, vi)

    @pl.when(j == NJ - 1)
    def _extract():
        for t in range(K):
            cand_v[:, t * LANES:(t + 1) * LANES] = best_v[t]
            cand_i[:, t * LANES:(t + 1) * LANES] = best_i[t]
        iota = lax.broadcasted_iota(jnp.int32, (BS_T, K * LANES), 1)
        for t in range(K):
            cv = cand_v[...]
            m = jnp.max(cv, axis=1, keepdims=True)
            hit = cv == m
            pos = jnp.min(jnp.where(hit, iota, BIG), axis=1, keepdims=True)
            sel = iota == pos
            win_i = jnp.sum(jnp.where(sel, cand_i[...], 0), axis=1, keepdims=True)
            out_v_ref[:, pl.ds(t, 1)] = m
            out_i_ref[:, pl.ds(t, 1)] = win_i
            cand_v[...] = jnp.where(sel, NEG, cv)


_NW = 32
_BPW = B // _NW
_SC_MESH = plsc.VectorSubcoreMesh(core_axis_name="c", subcore_axis_name="s")


@functools.partial(
    pl.kernel,
    mesh=_SC_MESH,
    out_type=jax.ShapeDtypeStruct((B, DIM), jnp.float32),
    scratch_types=[
        pltpu.VMEM((_BPW,), jnp.int32),
        pltpu.VMEM((_BPW,), jnp.int32),
        pltpu.VMEM((_BPW, DIM), jnp.float32),
        pltpu.VMEM((_BPW, DIM), jnp.float32),
        pltpu.SemaphoreType.DMA,
    ],
    compiler_params=pltpu.CompilerParams(use_tc_tiling_on_sc=False),
)
def _hr_gather(head_hbm, rel_hbm, ent_hbm, relemb_hbm, hr_hbm,
               hidx_v, ridx_v, hrows_v, rrows_v, sem):
    wid = lax.axis_index("s") * 2 + lax.axis_index("c")
    base = wid * _BPW
    pltpu.sync_copy(head_hbm.at[pl.ds(base, _BPW)], hidx_v)
    pltpu.sync_copy(rel_hbm.at[pl.ds(base, _BPW)], ridx_v)
    pltpu.async_copy(ent_hbm.at[hidx_v], hrows_v, sem).wait()
    pltpu.async_copy(relemb_hbm.at[ridx_v], rrows_v, sem).wait()
    for r in range(_BPW):
        for c in range(DIM // 16):
            sl = pl.ds(c * 16, 16)
            hrows_v[r, sl] = hrows_v[r, sl] * rrows_v[r, sl]
    pltpu.sync_copy(hrows_v, hr_hbm.at[pl.ds(base, _BPW)])


def kernel(relation, head, entity_embedding, relation_embedding):
    rel = relation.reshape(-1)
    hd = head.reshape(-1)
    hr = _hr_gather(hd, rel, entity_embedding, relation_embedding)

    e_pad = jnp.pad(entity_embedding, ((0, ENT_PAD - N_ENT), (0, 0)))

    out_v, out_i = pl.pallas_call(
        _topk_body,
        grid=(B // BS_T, NJ),
        in_specs=[
            pl.BlockSpec((BS_T, DIM), lambda i, j: (i, 0)),
            pl.BlockSpec((ENT_T, DIM), lambda i, j: (j, 0)),
        ],
        out_specs=[
            pl.BlockSpec((BS_T, LANES), lambda i, j: (i, 0)),
            pl.BlockSpec((BS_T, LANES), lambda i, j: (i, 0)),
        ],
        out_shape=[
            jax.ShapeDtypeStruct((B, LANES), jnp.float32),
            jax.ShapeDtypeStruct((B, LANES), jnp.int32),
        ],
        scratch_shapes=[
            pltpu.VMEM((K, BS_T, LANES), jnp.float32),
            pltpu.VMEM((K, BS_T, LANES), jnp.int32),
            pltpu.VMEM((BS_T, K * LANES), jnp.float32),
            pltpu.VMEM((BS_T, K * LANES), jnp.int32),
        ],
    )(hr, e_pad)

    return out_v[:, :K], out_i[:, :K]


# speedup vs baseline: 1.3428x; 1.3428x over previous
import functools

import jax
import jax.numpy as jnp
from jax import lax
from jax.experimental import pallas as pl
from jax.experimental.pallas import tpu as pltpu
from jax.experimental.pallas import tpu_sc as plsc

K = 10
N_ENT = 100000
N_REL = 1000
DIM = 64
B = 1024

BS_T = 256
LANES = 128
CELL = 8
ENT_T = CELL * LANES
ENT_PAD = 100352
NJ = ENT_PAD // ENT_T
N_CELLS = NJ * LANES
K_OUT = 16
NEG = -3.0e38
BIG = 1 << 30


_NW = 32
_BPW = B // _NW


@functools.lru_cache(maxsize=None)
def _hr_gather_kernel():
    return functools.partial(
        pl.kernel,
        mesh=plsc.VectorSubcoreMesh(core_axis_name="c", subcore_axis_name="s"),
        out_type=jax.ShapeDtypeStruct((B, DIM), jnp.float32),
        scratch_types=[
            pltpu.VMEM((_BPW,), jnp.int32),
            pltpu.VMEM((_BPW,), jnp.int32),
            pltpu.VMEM((_BPW, DIM), jnp.float32),
            pltpu.VMEM((_BPW, DIM), jnp.float32),
            pltpu.SemaphoreType.DMA,
        ],
        compiler_params=pltpu.CompilerParams(use_tc_tiling_on_sc=False),
    )(_hr_gather)


def _hr_gather(head_hbm, rel_hbm, ent_hbm, relemb_hbm, hr_hbm,
               hidx_v, ridx_v, hrows_v, rrows_v, sem):
    wid = lax.axis_index("s") * 2 + lax.axis_index("c")
    base = wid * _BPW
    pltpu.sync_copy(head_hbm.at[pl.ds(base, _BPW)], hidx_v)
    pltpu.sync_copy(rel_hbm.at[pl.ds(base, _BPW)], ridx_v)
    pltpu.async_copy(ent_hbm.at[hidx_v], hrows_v, sem).wait()
    pltpu.async_copy(relemb_hbm.at[ridx_v], rrows_v, sem).wait()
    for r in range(_BPW):
        for c in range(DIM // 16):
            sl = pl.ds(c * 16, 16)
            hrows_v[r, sl] = hrows_v[r, sl] * rrows_v[r, sl]
    pltpu.sync_copy(hrows_v, hr_hbm.at[pl.ds(base, _BPW)])


def _cell_topk_body(hr_ref, e_ref, out_i_ref, best_v, best_i, cand_v, cand_i):
    j = pl.program_id(1)

    @pl.when(j == 0)
    def _init():
        for t in range(K):
            best_v[t] = jnp.full((BS_T, LANES), NEG, jnp.float32)
            best_i[t] = jnp.zeros((BS_T, LANES), jnp.int32)

    hr = hr_ref[...]
    e = e_ref[...]
    scores = lax.dot_general(
        hr, e, (((1,), (1,)), ((), ())),
        preferred_element_type=jnp.float32,
    )

    lane_iota = lax.broadcasted_iota(jnp.int32, (BS_T, LANES), 1)
    cm = None
    for c in range(CELL):
        sl = scores[:, c * LANES:(c + 1) * LANES]
        ent = lane_iota + (j * ENT_T + c * LANES)
        sl = jnp.where(ent < N_ENT, sl, NEG)
        cm = sl if cm is None else jnp.maximum(cm, sl)

    v = cm
    vi = lane_iota + j * LANES
    for t in range(K):
        bv = best_v[t]
        bi = best_i[t]
        gt = v > bv
        best_v[t] = jnp.where(gt, v, bv)
        best_i[t] = jnp.where(gt, vi, bi)
        v = jnp.where(gt, bv, v)
        vi = jnp.where(gt, bi, vi)

    @pl.when(j == NJ - 1)
    def _extract():
        for t in range(K):
            cand_v[:, t * LANES:(t + 1) * LANES] = best_v[t]
            cand_i[:, t * LANES:(t + 1) * LANES] = best_i[t]
        iota = lax.broadcasted_iota(jnp.int32, (BS_T, K * LANES), 1)
        for t in range(K_OUT):
            cv = cand_v[...]
            m = jnp.max(cv, axis=1, keepdims=True)
            hit = cv == m
            pos = jnp.min(jnp.where(hit, iota, BIG), axis=1, keepdims=True)
            sel = iota == pos
            win_i = jnp.sum(jnp.where(sel, cand_i[...], 0), axis=1, keepdims=True)
            out_i_ref[:, pl.ds(t, 1)] = win_i
            cand_v[...] = jnp.where(sel, NEG, cv)


@functools.lru_cache(maxsize=None)
def _expand_gather_kernel():
    return functools.partial(
        pl.kernel,
        mesh=plsc.VectorSubcoreMesh(core_axis_name="c", subcore_axis_name="s"),
        out_type=[
            jax.ShapeDtypeStruct((B * LANES,), jnp.int32),
            jax.ShapeDtypeStruct((B * LANES, DIM), jnp.float32),
        ],
        scratch_types=[
            pltpu.VMEM((_BPW * LANES,), jnp.int32),
            pltpu.VMEM((LANES,), jnp.int32),
            pltpu.VMEM((_BPW * LANES,), jnp.int32),
            pltpu.VMEM((LANES, DIM), jnp.float32),
            pltpu.SemaphoreType.DMA,
        ],
        compiler_params=pltpu.CompilerParams(use_tc_tiling_on_sc=False),
    )(_expand_gather)


def _expand_gather(cells_hbm, ent_hbm, ids_hbm, g_hbm,
                   cells_v, idx_v, ids_v, erows_v, sem):
    wid = lax.axis_index("s") * 2 + lax.axis_index("c")
    base = wid * _BPW
    pltpu.sync_copy(cells_hbm.at[pl.ds(base * LANES, _BPW * LANES)], cells_v)

    def _row(r, carry):
        c16 = cells_v[pl.ds(r * LANES, 16)]
        g = c16 >> 7
        l = c16 & 127
        for grp in range(CELL):
            e = g * (CELL * LANES) + grp * LANES + l
            ids_v[pl.ds(r * LANES + grp * 16, 16)] = e
            idx_v[pl.ds(grp * 16, 16)] = jnp.minimum(e, N_ENT - 1)
        pltpu.async_copy(ent_hbm.at[idx_v], erows_v, sem).wait()
        pltpu.sync_copy(erows_v, g_hbm.at[pl.ds((base + r) * LANES, LANES)])
        return carry

    lax.fori_loop(0, _BPW, _row, 0)
    pltpu.sync_copy(ids_v, ids_hbm.at[pl.ds(base * LANES, _BPW * LANES)])


ROWS_T = 16


def _rescore_body(hr_ref, g_ref, ids_ref, out_v_ref, out_i_ref):
    hrb = hr_ref[...]
    gb = g_ref[...]
    s = lax.dot_general(
        hrb, gb, (((1,), (1,)), ((), ())),
        preferred_element_type=jnp.float32,
    )
    cand = jnp.concatenate(
        [s[i:i + 1, i * LANES:(i + 1) * LANES] for i in range(ROWS_T)], axis=0)
    ids_b = ids_ref[...]
    cand = jnp.where(ids_b < N_ENT, cand, NEG)
    iota = lax.broadcasted_iota(jnp.int32, (ROWS_T, LANES), 1)
    for t in range(K):
        m = jnp.max(cand, axis=1, keepdims=True)
        hit = cand == m
        pos = jnp.min(jnp.where(hit, iota, BIG), axis=1, keepdims=True)
        sel = iota == pos
        win_i = jnp.sum(jnp.where(sel, ids_b, 0), axis=1, keepdims=True)
        out_v_ref[:, pl.ds(t, 1)] = m
        out_i_ref[:, pl.ds(t, 1)] = win_i
        cand = jnp.where(sel, NEG, cand)


def kernel(relation, head, entity_embedding, relation_embedding):
    rel = relation.reshape(-1)
    hd = head.reshape(-1)
    hr = _hr_gather_kernel()(hd, rel, entity_embedding, relation_embedding)

    e_pad = jnp.pad(entity_embedding, ((0, ENT_PAD - N_ENT), (0, 0)))

    cells = pl.pallas_call(
        _cell_topk_body,
        grid=(B // BS_T, NJ),
        in_specs=[
            pl.BlockSpec((BS_T, DIM), lambda i, j: (i, 0)),
            pl.BlockSpec((ENT_T, DIM), lambda i, j: (j, 0)),
        ],
        out_specs=pl.BlockSpec((BS_T, LANES), lambda i, j: (i, 0)),
        out_shape=jax.ShapeDtypeStruct((B, LANES), jnp.int32),
        scratch_shapes=[
            pltpu.VMEM((K, BS_T, LANES), jnp.float32),
            pltpu.VMEM((K, BS_T, LANES), jnp.int32),
            pltpu.VMEM((BS_T, K * LANES), jnp.float32),
            pltpu.VMEM((BS_T, K * LANES), jnp.int32),
        ],
    )(hr, e_pad)

    ids_flat, g_flat = _expand_gather_kernel()(cells.reshape(-1), entity_embedding)

    out_v, out_i = pl.pallas_call(
        _rescore_body,
        grid=(B // ROWS_T,),
        in_specs=[
            pl.BlockSpec((ROWS_T, DIM), lambda i: (i, 0)),
            pl.BlockSpec((ROWS_T * LANES, DIM), lambda i: (i, 0)),
            pl.BlockSpec((ROWS_T, LANES), lambda i: (i, 0)),
        ],
        out_specs=[
            pl.BlockSpec((ROWS_T, LANES), lambda i: (i, 0)),
            pl.BlockSpec((ROWS_T, LANES), lambda i: (i, 0)),
        ],
        out_shape=[
            jax.ShapeDtypeStruct((B, LANES), jnp.float32),
            jax.ShapeDtypeStruct((B, LANES), jnp.int32),
        ],
    )(hr, g_flat, ids_flat.reshape(B, LANES))

    return out_v[:, :K], out_i[:, :K]


# speedup vs baseline: 1.3895x; 1.0348x over previous
import functools

import jax
import jax.numpy as jnp
from jax import lax
from jax.experimental import pallas as pl
from jax.experimental.pallas import tpu as pltpu
from jax.experimental.pallas import tpu_sc as plsc

K = 10
N_ENT = 100000
N_REL = 1000
DIM = 64
B = 1024

BS_T = 256
LANES = 128
CELL = 16
ENT_T = CELL * LANES
ENT_PAD = 100352
NJ = ENT_PAD // ENT_T
N_CELLS = NJ * LANES
K_OUT = 16
NEG = -3.0e38
BIG = 1 << 30


_NW = 32
_BPW = B // _NW


@functools.lru_cache(maxsize=None)
def _hr_gather_kernel():
    return functools.partial(
        pl.kernel,
        mesh=plsc.VectorSubcoreMesh(core_axis_name="c", subcore_axis_name="s"),
        out_type=jax.ShapeDtypeStruct((B, DIM), jnp.float32),
        scratch_types=[
            pltpu.VMEM((_BPW,), jnp.int32),
            pltpu.VMEM((_BPW,), jnp.int32),
            pltpu.VMEM((_BPW, DIM), jnp.float32),
            pltpu.VMEM((_BPW, DIM), jnp.float32),
            pltpu.SemaphoreType.DMA,
        ],
        compiler_params=pltpu.CompilerParams(use_tc_tiling_on_sc=False),
    )(_hr_gather)


def _hr_gather(head_hbm, rel_hbm, ent_hbm, relemb_hbm, hr_hbm,
               hidx_v, ridx_v, hrows_v, rrows_v, sem):
    wid = lax.axis_index("s") * 2 + lax.axis_index("c")
    base = wid * _BPW
    pltpu.sync_copy(head_hbm.at[pl.ds(base, _BPW)], hidx_v)
    pltpu.sync_copy(rel_hbm.at[pl.ds(base, _BPW)], ridx_v)
    pltpu.async_copy(ent_hbm.at[hidx_v], hrows_v, sem).wait()
    pltpu.async_copy(relemb_hbm.at[ridx_v], rrows_v, sem).wait()
    for r in range(_BPW):
        for c in range(DIM // 16):
            sl = pl.ds(c * 16, 16)
            hrows_v[r, sl] = hrows_v[r, sl] * rrows_v[r, sl]
    pltpu.sync_copy(hrows_v, hr_hbm.at[pl.ds(base, _BPW)])


def _cell_topk_body(hr_ref, e_ref, out_i_ref, best_v, best_i, cand_v, cand_i):
    j = pl.program_id(1)

    @pl.when(j == 0)
    def _init():
        for t in range(K):
            best_v[t] = jnp.full((BS_T, LANES), NEG, jnp.float32)
            best_i[t] = jnp.zeros((BS_T, LANES), jnp.int32)

    hr = hr_ref[...]
    e = e_ref[...]
    scores = lax.dot_general(
        hr, e, (((1,), (1,)), ((), ())),
        preferred_element_type=jnp.float32,
    )

    lane_iota = lax.broadcasted_iota(jnp.int32, (BS_T, LANES), 1)
    cm = None
    for c in range(CELL):
        sl = scores[:, c * LANES:(c + 1) * LANES]
        ent = lane_iota + (j * ENT_T + c * LANES)
        sl = jnp.where(ent < N_ENT, sl, NEG)
        cm = sl if cm is None else jnp.maximum(cm, sl)

    v = cm
    vi = lane_iota + j * LANES
    for t in range(K):
        bv = best_v[t]
        bi = best_i[t]
        gt = v > bv
        best_v[t] = jnp.where(gt, v, bv)
        best_i[t] = jnp.where(gt, vi, bi)
        v = jnp.where(gt, bv, v)
        vi = jnp.where(gt, bi, vi)

    @pl.when(j == NJ - 1)
    def _extract():
        for t in range(K):
            cand_v[:, t * LANES:(t + 1) * LANES] = best_v[t]
            cand_i[:, t * LANES:(t + 1) * LANES] = best_i[t]
        iota = lax.broadcasted_iota(jnp.int32, (BS_T, K * LANES), 1)
        for t in range(K_OUT):
            cv = cand_v[...]
            m = jnp.max(cv, axis=1, keepdims=True)
            hit = cv == m
            pos = jnp.min(jnp.where(hit, iota, BIG), axis=1, keepdims=True)
            sel = iota == pos
            win_i = jnp.sum(jnp.where(sel, cand_i[...], 0), axis=1, keepdims=True)
            out_i_ref[:, pl.ds(t, 1)] = win_i
            cand_v[...] = jnp.where(sel, NEG, cv)


EXPN = K_OUT * CELL


@functools.lru_cache(maxsize=None)
def _expand_gather_kernel():
    return functools.partial(
        pl.kernel,
        mesh=plsc.VectorSubcoreMesh(core_axis_name="c", subcore_axis_name="s"),
        out_type=[
            jax.ShapeDtypeStruct((B * EXPN,), jnp.int32),
            jax.ShapeDtypeStruct((B * EXPN, DIM), jnp.float32),
        ],
        scratch_types=[
            pltpu.VMEM((_BPW * LANES,), jnp.int32),
            pltpu.VMEM((LANES,), jnp.int32),
            pltpu.VMEM((_BPW * EXPN,), jnp.int32),
            pltpu.VMEM((EXPN, DIM), jnp.float32),
            pltpu.SemaphoreType.DMA,
        ],
        compiler_params=pltpu.CompilerParams(use_tc_tiling_on_sc=False),
    )(_expand_gather)


def _expand_gather(cells_hbm, ent_hbm, ids_hbm, g_hbm,
                   cells_v, idx_v, ids_v, erows_v, sem):
    wid = lax.axis_index("s") * 2 + lax.axis_index("c")
    base = wid * _BPW
    pltpu.sync_copy(cells_hbm.at[pl.ds(base * LANES, _BPW * LANES)], cells_v)

    def _row(r, carry):
        c16 = cells_v[pl.ds(r * LANES, 16)]
        g = c16 >> 7
        l = c16 & 127
        for half in range(EXPN // LANES):
            for sub in range(LANES // 16):
                grp = half * (LANES // 16) + sub
                e = g * (CELL * LANES) + grp * LANES + l
                ids_v[pl.ds(r * EXPN + grp * 16, 16)] = e
                idx_v[pl.ds(sub * 16, 16)] = jnp.minimum(e, N_ENT - 1)
            pltpu.async_copy(
                ent_hbm.at[idx_v], erows_v.at[pl.ds(half * LANES, LANES)],
                sem).wait()
        pltpu.sync_copy(erows_v, g_hbm.at[pl.ds((base + r) * EXPN, EXPN)])
        return carry

    lax.fori_loop(0, _BPW, _row, 0)
    pltpu.sync_copy(ids_v, ids_hbm.at[pl.ds(base * EXPN, _BPW * EXPN)])


ROWS_T = 16


def _rescore_body(hr_ref, g_ref, ids_ref, out_v_ref, out_i_ref):
    hrb = hr_ref[...]
    gb = g_ref[...]
    s = lax.dot_general(
        hrb, gb, (((1,), (1,)), ((), ())),
        preferred_element_type=jnp.float32,
    )
    cand = jnp.concatenate(
        [s[i:i + 1, i * EXPN:(i + 1) * EXPN] for i in range(ROWS_T)], axis=0)
    ids_b = ids_ref[...]
    cand = jnp.where(ids_b < N_ENT, cand, NEG)
    iota = lax.broadcasted_iota(jnp.int32, (ROWS_T, EXPN), 1)
    for t in range(K):
        m = jnp.max(cand, axis=1, keepdims=True)
        hit = cand == m
        pos = jnp.min(jnp.where(hit, iota, BIG), axis=1, keepdims=True)
        sel = iota == pos
        win_i = jnp.sum(jnp.where(sel, ids_b, 0), axis=1, keepdims=True)
        out_v_ref[:, pl.ds(t, 1)] = m
        out_i_ref[:, pl.ds(t, 1)] = win_i
        cand = jnp.where(sel, NEG, cand)


def kernel(relation, head, entity_embedding, relation_embedding):
    rel = relation.reshape(-1)
    hd = head.reshape(-1)
    hr = _hr_gather_kernel()(hd, rel, entity_embedding, relation_embedding)

    e_pad = jnp.pad(entity_embedding, ((0, ENT_PAD - N_ENT), (0, 0)))

    cells = pl.pallas_call(
        _cell_topk_body,
        grid=(B // BS_T, NJ),
        in_specs=[
            pl.BlockSpec((BS_T, DIM), lambda i, j: (i, 0)),
            pl.BlockSpec((ENT_T, DIM), lambda i, j: (j, 0)),
        ],
        out_specs=pl.BlockSpec((BS_T, LANES), lambda i, j: (i, 0)),
        out_shape=jax.ShapeDtypeStruct((B, LANES), jnp.int32),
        scratch_shapes=[
            pltpu.VMEM((K, BS_T, LANES), jnp.float32),
            pltpu.VMEM((K, BS_T, LANES), jnp.int32),
            pltpu.VMEM((BS_T, K * LANES), jnp.float32),
            pltpu.VMEM((BS_T, K * LANES), jnp.int32),
        ],
    )(hr, e_pad)

    ids_flat, g_flat = _expand_gather_kernel()(cells.reshape(-1), entity_embedding)

    out_v, out_i = pl.pallas_call(
        _rescore_body,
        grid=(B // ROWS_T,),
        in_specs=[
            pl.BlockSpec((ROWS_T, DIM), lambda i: (i, 0)),
            pl.BlockSpec((ROWS_T * EXPN, DIM), lambda i: (i, 0)),
            pl.BlockSpec((ROWS_T, EXPN), lambda i: (i, 0)),
        ],
        out_specs=[
            pl.BlockSpec((ROWS_T, LANES), lambda i: (i, 0)),
            pl.BlockSpec((ROWS_T, LANES), lambda i: (i, 0)),
        ],
        out_shape=[
            jax.ShapeDtypeStruct((B, LANES), jnp.float32),
            jax.ShapeDtypeStruct((B, LANES), jnp.int32),
        ],
    )(hr, g_flat, ids_flat.reshape(B, EXPN))

    return out_v[:, :K], out_i[:, :K]
